# baseline (speedup 1.0000x reference)
import functools

import jax
import jax.numpy as jnp
from jax import lax
from jax.experimental import pallas as pl
from jax.experimental.pallas import tpu as pltpu
from jax.experimental.pallas import tpu_sc as plsc

NC, NS, L = 2, 16, 16
NW = NC * NS
B, DIM, H, W = 32, 64, 128, 128
HW = H * W
BLK = HW // L
M = 500
MP = 512
G = 4
NG = DIM // G
GMP = G * MP


@functools.partial(
    pl.kernel,
    out_type=(
        jax.ShapeDtypeStruct((NW, L), jnp.float32),
        jax.ShapeDtypeStruct((NW, L), jnp.float32),
    ),
    mesh=plsc.VectorSubcoreMesh(
        core_axis_name="c", subcore_axis_name="s",
        num_cores=NC, num_subcores=NS),
    compiler_params=pltpu.CompilerParams(
        needs_layout_passes=False, use_tc_tiling_on_sc=False),
    scratch_types=[
        pltpu.VMEM((MP,), jnp.int32),
        pltpu.VMEM((GMP,), jnp.int32),
        pltpu.VMEM((GMP,), jnp.int32),
        pltpu.VMEM((GMP,), jnp.float32),
        pltpu.VMEM((MP, DIM), jnp.float32),
        pltpu.VMEM((2, GMP, L), jnp.float32),
        pltpu.VMEM((L,), jnp.float32),
        pltpu.VMEM((L,), jnp.float32),
        pltpu.SemaphoreType.DMA,
        pltpu.SemaphoreType.DMA,
        pltpu.SemaphoreType.DMA,
    ],
)
def _sc_loss(outblk, indf, maskf, tgtf, loss_out, num_out,
             ind_v, cix_v, off_v, mask_v, tgt_v, blk_v, acc_v, nacc_v,
             sem0, sem1, semt):
    w = lax.axis_index("s") * NC + lax.axis_index("c")
    base = w * DIM

    tgt_cp = pltpu.async_copy(tgtf.at[w], tgt_v.at[pl.ds(0, M)], semt)
    pltpu.sync_copy(indf.at[w], ind_v)
    pltpu.sync_copy(maskf.at[w], mask_v.at[pl.ds(0, MP)])

    def split_body(j, nacc):
        iv = ind_v[pl.ds(j * L, L)]
        ish = iv >> 4
        off = iv & 15
        mk = mask_v[pl.ds(j * L, L)]
        for k in range(G):
            cix_v[pl.ds(k * MP + j * L, L)] = ish + (k * BLK)
            off_v[pl.ds(k * MP + j * L, L)] = off
            if k:
                mask_v[pl.ds(k * MP + j * L, L)] = mk
        return nacc + mk

    nacc = lax.fori_loop(0, MP // L, split_body,
                         jnp.zeros((L,), jnp.float32), unroll=2)

    def gather_group(g, buf, sem):
        table = outblk.at[pl.ds((base + g * G) * BLK, G * BLK)]
        return pltpu.async_copy(table.at[cix_v], blk_v.at[buf], sem)

    def wait_group(g, buf, sem):
        table = outblk.at[pl.ds((base + g * G) * BLK, G * BLK)]
        pltpu.make_async_copy(table.at[cix_v], blk_v.at[buf], sem).wait()

    rowids = lax.iota(jnp.int32, L)

    def compute_group(buf, g, acc):
        blk = blk_v.at[buf]

        for k in range(G):
            c = g * G + k
            cvec = jnp.full((L,), c, jnp.int32)

            def m_body(j, acc):
                sids = j * L + rowids
                p = plsc.load_gather(blk, [k * MP + sids,
                                           off_v[pl.ds(k * MP + j * L, L)]])
                traw = plsc.load_gather(tgt_v, [sids, cvec])
                mk = mask_v[pl.ds(j * L, L)]
                t = jnp.where(mk != 0.0, traw, 0.0)
                d = (p - t) * mk
                a = jnp.abs(d)
                m1 = jnp.minimum(a, 1.0)
                return acc + (0.5 * m1 * m1 - 1.0 + jnp.maximum(a, 1.0))

            acc = lax.fori_loop(0, MP // L, m_body, acc, unroll=4)
        return acc

    gather_group(0, 0, sem0)
    gather_group(1, 1, sem1)
    tgt_cp.wait()

    def step(i, acc):
        g0 = 2 * i
        wait_group(g0, 0, sem0)
        acc = compute_group(0, g0, acc)

        @pl.when(i < NG // 2 - 1)
        def _():
            gather_group(g0 + 2, 0, sem0)

        wait_group(g0 + 1, 1, sem1)
        acc = compute_group(1, g0 + 1, acc)

        @pl.when(i < NG // 2 - 1)
        def _():
            gather_group(g0 + 3, 1, sem1)

        return acc

    acc = lax.fori_loop(0, NG // 2, step, jnp.zeros((L,), jnp.float32))

    acc_v[...] = acc
    nacc_v[...] = nacc
    pltpu.sync_copy(acc_v, loss_out.at[w])
    pltpu.sync_copy(nacc_v, num_out.at[w])


def kernel(output, mask, ind, target):
    outblk = output.reshape(B * DIM * BLK, L)
    ind32 = jnp.pad(ind.astype(jnp.int32), ((0, 0), (0, MP - M)))
    maskf = jnp.pad(mask.astype(jnp.float32), ((0, 0), (0, MP - M)))
    loss_p, num_p = _sc_loss(outblk, ind32, maskf, target)
    return jnp.sum(loss_p) / (jnp.sum(num_p) + 0.0001)


# speedup vs baseline: 1.2406x; 1.2406x over previous
import functools

import jax
import jax.numpy as jnp
from jax import lax
from jax.experimental import pallas as pl
from jax.experimental.pallas import tpu as pltpu
from jax.experimental.pallas import tpu_sc as plsc

NC, NS, L = 2, 16, 16
NW = NC * NS
B, DIM, H, W = 32, 64, 128, 128
HW = H * W
M = 500
MP = 512
G = 4
NG = DIM // G
GMP = G * MP


@functools.partial(
    pl.kernel,
    out_type=(
        jax.ShapeDtypeStruct((NW, L), jnp.float32),
        jax.ShapeDtypeStruct((NW, L), jnp.float32),
    ),
    mesh=plsc.VectorSubcoreMesh(
        core_axis_name="c", subcore_axis_name="s",
        num_cores=NC, num_subcores=NS),
    compiler_params=pltpu.CompilerParams(
        needs_layout_passes=False, use_tc_tiling_on_sc=False),
    scratch_types=[
        pltpu.VMEM((MP,), jnp.int32),
        pltpu.VMEM((GMP,), jnp.int32),
        pltpu.VMEM((GMP,), jnp.float32),
        pltpu.VMEM((DIM * MP,), jnp.float32),
        pltpu.VMEM((2, GMP), jnp.float32),
        pltpu.VMEM((L,), jnp.float32),
        pltpu.VMEM((L,), jnp.float32),
        pltpu.SemaphoreType.DMA,
        pltpu.SemaphoreType.DMA,
        pltpu.SemaphoreType.DMA,
    ],
)
def _sc_loss(outel, indf, maskf, tgtf, loss_out, num_out,
             ind_v, cix_v, mask_v, tgt_v, prd_v, acc_v, nacc_v,
             sem0, sem1, semt):
    w = lax.axis_index("s") * NC + lax.axis_index("c")
    base = w * DIM

    tgt_cp = pltpu.async_copy(tgtf.at[w], tgt_v, semt)
    pltpu.sync_copy(indf.at[w], ind_v)
    pltpu.sync_copy(maskf.at[w], mask_v.at[pl.ds(0, MP)])

    def split_body(j, nacc):
        iv = ind_v[pl.ds(j * L, L)]
        mk = mask_v[pl.ds(j * L, L)]
        for k in range(G):
            cix_v[pl.ds(k * MP + j * L, L)] = iv + (k * HW)
            if k:
                mask_v[pl.ds(k * MP + j * L, L)] = mk
        return nacc + mk

    nacc = lax.fori_loop(0, MP // L, split_body,
                         jnp.zeros((L,), jnp.float32), unroll=2)

    def gather_group(g, buf, sem):
        table = outel.at[pl.ds((base + g * G) * HW, G * HW)]
        return pltpu.async_copy(table.at[cix_v], prd_v.at[buf], sem)

    def wait_group(g, buf, sem):
        table = outel.at[pl.ds((base + g * G) * HW, G * HW)]
        pltpu.make_async_copy(table.at[cix_v], prd_v.at[buf], sem).wait()

    def compute_group(buf, g, acc):
        tbase = g * GMP

        def m_body(q, acc):
            p = prd_v[buf, pl.ds(q * L, L)]
            t = tgt_v[pl.ds(tbase + q * L, L)]
            mk = mask_v[pl.ds(q * L, L)]
            d = (p - t) * mk
            a = jnp.abs(d)
            m1 = jnp.minimum(a, 1.0)
            return acc + (0.5 * m1 * m1 - 1.0 + jnp.maximum(a, 1.0))

        return lax.fori_loop(0, GMP // L, m_body, acc, unroll=4)

    gather_group(0, 0, sem0)
    gather_group(1, 1, sem1)
    tgt_cp.wait()

    def step(i, acc):
        g0 = 2 * i
        wait_group(g0, 0, sem0)
        acc = compute_group(0, g0, acc)

        @pl.when(i < NG // 2 - 1)
        def _():
            gather_group(g0 + 2, 0, sem0)

        wait_group(g0 + 1, 1, sem1)
        acc = compute_group(1, g0 + 1, acc)

        @pl.when(i < NG // 2 - 1)
        def _():
            gather_group(g0 + 3, 1, sem1)

        return acc

    acc = lax.fori_loop(0, NG // 2, step, jnp.zeros((L,), jnp.float32))

    acc_v[...] = acc
    nacc_v[...] = nacc
    pltpu.sync_copy(acc_v, loss_out.at[w])
    pltpu.sync_copy(nacc_v, num_out.at[w])


def kernel(output, mask, ind, target):
    outel = output.reshape(B * DIM * HW)
    ind32 = jnp.pad(ind.astype(jnp.int32), ((0, 0), (0, MP - M)))
    maskf = jnp.pad(mask.astype(jnp.float32), ((0, 0), (0, MP - M)))
    tgtT = jnp.pad(jnp.transpose(target, (0, 2, 1)),
                   ((0, 0), (0, 0), (0, MP - M)))
    tgtflat = tgtT.reshape(B, DIM * MP)
    loss_p, num_p = _sc_loss(outel, ind32, maskf, tgtflat)
    return jnp.sum(loss_p) / (jnp.sum(num_p) + 0.0001)


# speedup vs baseline: 1.2499x; 1.0075x over previous
import functools

import jax
import jax.numpy as jnp
from jax import lax
from jax.experimental import pallas as pl
from jax.experimental.pallas import tpu as pltpu
from jax.experimental.pallas import tpu_sc as plsc

NC, NS, L = 2, 16, 16
NW = NC * NS
B, DIM, H, W = 32, 64, 128, 128
HW = H * W
M = 500
MP = 512
G = 8
NG = DIM // G
GMP = G * MP


@functools.partial(
    pl.kernel,
    out_type=(
        jax.ShapeDtypeStruct((NW, L), jnp.float32),
        jax.ShapeDtypeStruct((NW, L), jnp.float32),
    ),
    mesh=plsc.VectorSubcoreMesh(
        core_axis_name="c", subcore_axis_name="s",
        num_cores=NC, num_subcores=NS),
    compiler_params=pltpu.CompilerParams(
        needs_layout_passes=False, use_tc_tiling_on_sc=False),
    scratch_types=[
        pltpu.VMEM((MP,), jnp.int32),
        pltpu.VMEM((GMP,), jnp.int32),
        pltpu.VMEM((GMP,), jnp.float32),
        pltpu.VMEM((DIM * MP,), jnp.float32),
        pltpu.VMEM((2, GMP), jnp.float32),
        pltpu.VMEM((L,), jnp.float32),
        pltpu.VMEM((L,), jnp.float32),
        pltpu.SemaphoreType.DMA,
        pltpu.SemaphoreType.DMA,
        pltpu.SemaphoreType.DMA,
    ],
)
def _sc_loss(outel, indf, maskf, tgtf, loss_out, num_out,
             ind_v, cix_v, mask_v, tgt_v, prd_v, acc_v, nacc_v,
             sem0, sem1, semt):
    w = lax.axis_index("s") * NC + lax.axis_index("c")
    base = w * DIM

    tgt_cp = pltpu.async_copy(tgtf.at[w], tgt_v, semt)
    pltpu.sync_copy(indf.at[w], ind_v)
    pltpu.sync_copy(maskf.at[w], mask_v.at[pl.ds(0, MP)])

    def split_body(j, nacc):
        iv = ind_v[pl.ds(j * L, L)]
        mk = mask_v[pl.ds(j * L, L)]
        for k in range(G):
            cix_v[pl.ds(k * MP + j * L, L)] = iv + (k * HW)
            if k:
                mask_v[pl.ds(k * MP + j * L, L)] = mk
        return nacc + mk

    nacc = lax.fori_loop(0, MP // L, split_body,
                         jnp.zeros((L,), jnp.float32), unroll=2)

    def gather_group(g, buf, sem):
        table = outel.at[pl.ds((base + g * G) * HW, G * HW)]
        return pltpu.async_copy(table.at[cix_v], prd_v.at[buf], sem)

    def wait_group(g, buf, sem):
        table = outel.at[pl.ds((base + g * G) * HW, G * HW)]
        pltpu.make_async_copy(table.at[cix_v], prd_v.at[buf], sem).wait()

    def compute_group(buf, g, acc):
        tbase = g * GMP

        def m_body(q, acc):
            p = prd_v[buf, pl.ds(q * L, L)]
            t = tgt_v[pl.ds(tbase + q * L, L)]
            mk = mask_v[pl.ds(q * L, L)]
            d = (p - t) * mk
            a = jnp.abs(d)
            m1 = jnp.minimum(a, 1.0)
            return acc + (0.5 * m1 * m1 - 1.0 + jnp.maximum(a, 1.0))

        return lax.fori_loop(0, GMP // L, m_body, acc, unroll=4)

    gather_group(0, 0, sem0)
    gather_group(1, 1, sem1)
    tgt_cp.wait()

    def step(i, acc):
        g0 = 2 * i
        wait_group(g0, 0, sem0)
        acc = compute_group(0, g0, acc)

        @pl.when(i < NG // 2 - 1)
        def _():
            gather_group(g0 + 2, 0, sem0)

        wait_group(g0 + 1, 1, sem1)
        acc = compute_group(1, g0 + 1, acc)

        @pl.when(i < NG // 2 - 1)
        def _():
            gather_group(g0 + 3, 1, sem1)

        return acc

    acc = lax.fori_loop(0, NG // 2, step, jnp.zeros((L,), jnp.float32))

    acc_v[...] = acc
    nacc_v[...] = nacc
    pltpu.sync_copy(acc_v, loss_out.at[w])
    pltpu.sync_copy(nacc_v, num_out.at[w])


def kernel(output, mask, ind, target):
    outel = output.reshape(B * DIM * HW)
    ind32 = jnp.pad(ind.astype(jnp.int32), ((0, 0), (0, MP - M)))
    maskf = jnp.pad(mask.astype(jnp.float32), ((0, 0), (0, MP - M)))
    tgtT = jnp.pad(jnp.transpose(target, (0, 2, 1)),
                   ((0, 0), (0, 0), (0, MP - M)))
    tgtflat = tgtT.reshape(B, DIM * MP)
    loss_p, num_p = _sc_loss(outel, ind32, maskf, tgtflat)
    return jnp.sum(loss_p) / (jnp.sum(num_p) + 0.0001)
